# baseline (speedup 1.0000x reference)
import functools

import jax
import jax.numpy as jnp
from jax import lax
from jax.experimental import pallas as pl
from jax.experimental.pallas import tpu as pltpu
from jax.experimental.pallas import tpu_sc as plsc

_NUM_WORKERS = 32
_CHUNK = 128
_GROUP = 4
_LANES = 16


@functools.lru_cache(maxsize=None)
def _build_detile(num_rows: int, embed_dim: int):
    assert embed_dim == 32 and num_rows == 1000000
    jt_n = embed_dim // 8
    full_p = num_rows // 128
    tail = num_rows - full_p * 128
    per_w = full_p // _NUM_WORKERS
    n_left = full_p - per_w * _NUM_WORKERS
    assert per_w % 2 == 0
    mesh = plsc.VectorSubcoreMesh(core_axis_name="c", subcore_axis_name="s")

    @functools.partial(
        pl.kernel,
        out_type=jax.ShapeDtypeStruct((num_rows * embed_dim,), jnp.float32),
        mesh=mesh,
        scratch_types=[
            pltpu.VMEM((jt_n, 8, 128), jnp.float32),
            pltpu.VMEM((jt_n, 8, 128), jnp.float32),
            pltpu.VMEM((128 * embed_dim,), jnp.float32),
            pltpu.VMEM((128 * embed_dim,), jnp.float32),
            pltpu.SemaphoreType.DMA,
            pltpu.SemaphoreType.DMA,
            pltpu.SemaphoreType.DMA,
            pltpu.SemaphoreType.DMA,
        ],
        compiler_params=pltpu.CompilerParams(
            use_tc_tiling_on_sc=True, needs_layout_passes=False
        ),
    )
    def detile_kernel(wt_hbm, wtail_hbm, out_hbm, in_a, in_b, tr_a, tr_b,
                      sem_ia, sem_ib, sem_oa, sem_ob):
        num_cores = lax.axis_size("c")
        wid = lax.axis_index("s") * num_cores + lax.axis_index("c")
        lane32 = lax.iota(jnp.int32, _LANES) * embed_dim

        def fire_in(p, buf, sem):
            for t in range(jt_n):
                pltpu.async_copy(
                    wt_hbm.at[pl.ds(8 * t, 8), pl.ds(p * 128, 128)],
                    buf.at[t], sem,
                )

        def drain_in(buf, sem):
            for t in range(jt_n):
                pltpu.make_async_copy(
                    wt_hbm.at[pl.ds(0, 8), pl.ds(0, 128)], buf.at[t], sem
                ).wait()

        def transpose(buf, tr, csteps):
            for m in range(embed_dim):
                t, j = m // 8, m % 8
                for c in range(csteps):
                    v = buf[t, j, pl.ds(c * _LANES, _LANES)]
                    plsc.store_scatter(
                        tr, [lane32 + (c * _LANES * embed_dim + m)], v
                    )

        def write_out(p, tr, sem):
            pltpu.async_copy(
                tr, out_hbm.at[pl.ds(p * 128 * embed_dim, 128 * embed_dim)], sem
            )

        def drain_out(tr, sem):
            pltpu.make_async_copy(
                out_hbm.at[pl.ds(0, 128 * embed_dim)], tr, sem
            ).wait()

        fire_in(wid, in_a, sem_ia)

        def body(k, carry):
            p0 = wid + _NUM_WORKERS * 2 * k
            fire_in(p0 + _NUM_WORKERS, in_b, sem_ib)
            drain_in(in_a, sem_ia)

            @pl.when(k > 0)
            def _():
                drain_out(tr_a, sem_oa)

            transpose(in_a, tr_a, 8)
            write_out(p0, tr_a, sem_oa)

            @pl.when(k + 1 < per_w // 2)
            def _():
                fire_in(p0 + 2 * _NUM_WORKERS, in_a, sem_ia)

            drain_in(in_b, sem_ib)

            @pl.when(k > 0)
            def _():
                drain_out(tr_b, sem_ob)

            transpose(in_b, tr_b, 8)
            write_out(p0 + _NUM_WORKERS, tr_b, sem_ob)
            return carry

        lax.fori_loop(0, per_w // 2, body, 0)
        drain_out(tr_a, sem_oa)
        drain_out(tr_b, sem_ob)

        p_extra = wid + _NUM_WORKERS * per_w

        @pl.when(jnp.logical_and(wid < n_left, p_extra < full_p))
        def _():
            fire_in(p_extra, in_a, sem_ia)
            drain_in(in_a, sem_ia)
            transpose(in_a, tr_a, 8)
            write_out(p_extra, tr_a, sem_oa)
            drain_out(tr_a, sem_oa)

        if tail:
            @pl.when(p_extra == full_p)
            def _():
                pltpu.sync_copy(wtail_hbm, tr_a.at[pl.ds(0, tail * embed_dim)])
                pltpu.sync_copy(
                    tr_a.at[pl.ds(0, tail * embed_dim)],
                    out_hbm.at[pl.ds(full_p * 128 * embed_dim, tail * embed_dim)],
                )

    return detile_kernel


@functools.lru_cache(maxsize=None)
def _build_gather(batch: int, num_fields: int, embed_dim: int):
    assert batch % (_NUM_WORKERS * _CHUNK) == 0
    cols_per_w = batch // _NUM_WORKERS
    chunks_per_f = cols_per_w // _CHUNK
    steps = num_fields * chunks_per_f
    assert steps % (2 * _GROUP) == 0
    grows = _GROUP * _CHUNK
    mesh = plsc.VectorSubcoreMesh(core_axis_name="c", subcore_axis_name="s")

    @functools.partial(
        pl.kernel,
        out_type=jax.ShapeDtypeStruct((num_fields, batch, embed_dim), jnp.float32),
        mesh=mesh,
        scratch_types=[
            pltpu.VMEM((num_fields, cols_per_w), jnp.int32),
            pltpu.VMEM((grows, embed_dim), jnp.float32),
            pltpu.VMEM((grows, embed_dim), jnp.float32),
            pltpu.SemaphoreType.DMA,
            pltpu.SemaphoreType.DMA,
        ],
        compiler_params=pltpu.CompilerParams(use_tc_tiling_on_sc=False),
    )
    def gather_kernel(table_hbm, xt_hbm, out_hbm, idx_v, buf_a, buf_b, sem_a, sem_b):
        num_cores = lax.axis_size("c")
        wid = lax.axis_index("s") * num_cores + lax.axis_index("c")
        base = wid * cols_per_w
        pltpu.sync_copy(xt_hbm.at[:, pl.ds(base, cols_per_w)], idx_v)

        def fire(g, buf, sem):
            for k in range(_GROUP):
                s = g * _GROUP + k
                f = s // chunks_per_f
                c = s % chunks_per_f
                pltpu.async_copy(
                    table_hbm.at[idx_v.at[f, pl.ds(c * _CHUNK, _CHUNK)]],
                    buf.at[pl.ds(k * _CHUNK, _CHUNK)],
                    sem,
                )

        def drain_write(g, buf, sem):
            pltpu.make_async_copy(table_hbm.at[pl.ds(0, grows)], buf, sem).wait()
            f = (g * _GROUP) // chunks_per_f
            c = (g * _GROUP) % chunks_per_f
            pltpu.sync_copy(buf, out_hbm.at[f, pl.ds(base + c * _CHUNK, grows)])

        assert chunks_per_f % _GROUP == 0

        fire(0, buf_a, sem_a)

        def body(p, carry):
            g = 2 * p
            fire(g + 1, buf_b, sem_b)
            drain_write(g, buf_a, sem_a)

            @pl.when(p + 1 < steps // (2 * _GROUP))
            def _():
                fire(g + 2, buf_a, sem_a)

            drain_write(g + 1, buf_b, sem_b)
            return carry

        lax.fori_loop(0, steps // (2 * _GROUP), body, 0)

    return gather_kernel


def kernel(x, weight):
    batch, num_fields = x.shape
    num_rows, embed_dim = weight.shape
    tail_rows = num_rows - (num_rows // 128) * 128
    wtail = weight[num_rows - tail_rows:].reshape(tail_rows * embed_dim)
    flat = _build_detile(num_rows, embed_dim)(weight.T, wtail)
    wlin = flat.reshape(num_rows, embed_dim)
    out_t = _build_gather(batch, num_fields, embed_dim)(
        wlin, x.T.astype(jnp.int32)
    )
    return out_t.transpose(1, 0, 2)


# speedup vs baseline: 1.0608x; 1.0608x over previous
import functools

import jax
import jax.numpy as jnp
from jax import lax
from jax.experimental import pallas as pl
from jax.experimental.pallas import tpu as pltpu
from jax.experimental.pallas import tpu_sc as plsc

_NUM_WORKERS = 32
_CHUNK = 128
_GROUP = 4
_LANES = 16


@functools.lru_cache(maxsize=None)
def _build_detile(num_rows: int, embed_dim: int):
    assert embed_dim == 32 and num_rows == 1000000
    jt_n = embed_dim // 8
    full_p = num_rows // 128
    tail = num_rows - full_p * 128
    per_w = full_p // _NUM_WORKERS
    n_left = full_p - per_w * _NUM_WORKERS
    assert per_w % 2 == 0
    mesh = plsc.VectorSubcoreMesh(core_axis_name="c", subcore_axis_name="s")

    @functools.partial(
        pl.kernel,
        out_type=jax.ShapeDtypeStruct((num_rows * embed_dim,), jnp.float32),
        mesh=mesh,
        scratch_types=[
            pltpu.VMEM((jt_n, 8, 128), jnp.float32),
            pltpu.VMEM((jt_n, 8, 128), jnp.float32),
            pltpu.VMEM((128 * embed_dim,), jnp.float32),
            pltpu.VMEM((128 * embed_dim,), jnp.float32),
            pltpu.SemaphoreType.DMA,
            pltpu.SemaphoreType.DMA,
            pltpu.SemaphoreType.DMA,
            pltpu.SemaphoreType.DMA,
        ],
        compiler_params=pltpu.CompilerParams(
            use_tc_tiling_on_sc=True, needs_layout_passes=False
        ),
    )
    def detile_kernel(wt_hbm, wtail_hbm, out_hbm, in_a, in_b, tr_a, tr_b,
                      sem_ia, sem_ib, sem_oa, sem_ob):
        num_cores = lax.axis_size("c")
        wid = lax.axis_index("s") * num_cores + lax.axis_index("c")
        lane32 = lax.iota(jnp.int32, _LANES) * embed_dim

        def fire_in(p, buf, sem):
            for t in range(jt_n):
                pltpu.async_copy(
                    wt_hbm.at[pl.ds(8 * t, 8), pl.ds(p * 128, 128)],
                    buf.at[t], sem,
                )

        def drain_in(buf, sem):
            for t in range(jt_n):
                pltpu.make_async_copy(
                    wt_hbm.at[pl.ds(0, 8), pl.ds(0, 128)], buf.at[t], sem
                ).wait()

        def transpose(buf, tr, csteps):
            assert csteps == 8
            @plsc.parallel_loop(0, embed_dim * csteps, 1, unroll=8)
            def step(v):
                m = v >> 3
                c = v & 7
                t = m >> 3
                j = m & 7
                vec = buf[t, j, pl.ds(c * _LANES, _LANES)]
                plsc.store_scatter(tr, [lane32 + (c * _LANES * embed_dim + m)], vec)

        def write_out(p, tr, sem):
            pltpu.async_copy(
                tr, out_hbm.at[pl.ds(p * 128 * embed_dim, 128 * embed_dim)], sem
            )

        def drain_out(tr, sem):
            pltpu.make_async_copy(
                out_hbm.at[pl.ds(0, 128 * embed_dim)], tr, sem
            ).wait()

        fire_in(wid, in_a, sem_ia)

        def body(k, carry):
            p0 = wid + _NUM_WORKERS * 2 * k
            fire_in(p0 + _NUM_WORKERS, in_b, sem_ib)
            drain_in(in_a, sem_ia)

            @pl.when(k > 0)
            def _():
                drain_out(tr_a, sem_oa)

            transpose(in_a, tr_a, 8)
            write_out(p0, tr_a, sem_oa)

            @pl.when(k + 1 < per_w // 2)
            def _():
                fire_in(p0 + 2 * _NUM_WORKERS, in_a, sem_ia)

            drain_in(in_b, sem_ib)

            @pl.when(k > 0)
            def _():
                drain_out(tr_b, sem_ob)

            transpose(in_b, tr_b, 8)
            write_out(p0 + _NUM_WORKERS, tr_b, sem_ob)
            return carry

        lax.fori_loop(0, per_w // 2, body, 0)
        drain_out(tr_a, sem_oa)
        drain_out(tr_b, sem_ob)

        p_extra = wid + _NUM_WORKERS * per_w

        @pl.when(jnp.logical_and(wid < n_left, p_extra < full_p))
        def _():
            fire_in(p_extra, in_a, sem_ia)
            drain_in(in_a, sem_ia)
            transpose(in_a, tr_a, 8)
            write_out(p_extra, tr_a, sem_oa)
            drain_out(tr_a, sem_oa)

        if tail:
            @pl.when(p_extra == full_p)
            def _():
                pltpu.sync_copy(wtail_hbm, tr_a.at[pl.ds(0, tail * embed_dim)])
                pltpu.sync_copy(
                    tr_a.at[pl.ds(0, tail * embed_dim)],
                    out_hbm.at[pl.ds(full_p * 128 * embed_dim, tail * embed_dim)],
                )

    return detile_kernel


@functools.lru_cache(maxsize=None)
def _build_gather(batch: int, num_fields: int, embed_dim: int):
    assert batch % (_NUM_WORKERS * _CHUNK) == 0
    cols_per_w = batch // _NUM_WORKERS
    chunks_per_f = cols_per_w // _CHUNK
    steps = num_fields * chunks_per_f
    assert steps % (2 * _GROUP) == 0
    grows = _GROUP * _CHUNK
    mesh = plsc.VectorSubcoreMesh(core_axis_name="c", subcore_axis_name="s")

    @functools.partial(
        pl.kernel,
        out_type=jax.ShapeDtypeStruct((num_fields, batch, embed_dim), jnp.float32),
        mesh=mesh,
        scratch_types=[
            pltpu.VMEM((num_fields, cols_per_w), jnp.int32),
            pltpu.VMEM((grows, embed_dim), jnp.float32),
            pltpu.VMEM((grows, embed_dim), jnp.float32),
            pltpu.SemaphoreType.DMA,
            pltpu.SemaphoreType.DMA,
        ],
        compiler_params=pltpu.CompilerParams(use_tc_tiling_on_sc=False),
    )
    def gather_kernel(table_hbm, xt_hbm, out_hbm, idx_v, buf_a, buf_b, sem_a, sem_b):
        num_cores = lax.axis_size("c")
        wid = lax.axis_index("s") * num_cores + lax.axis_index("c")
        base = wid * cols_per_w
        pltpu.sync_copy(xt_hbm.at[:, pl.ds(base, cols_per_w)], idx_v)

        def fire(g, buf, sem):
            for k in range(_GROUP):
                s = g * _GROUP + k
                f = s // chunks_per_f
                c = s % chunks_per_f
                pltpu.async_copy(
                    table_hbm.at[idx_v.at[f, pl.ds(c * _CHUNK, _CHUNK)]],
                    buf.at[pl.ds(k * _CHUNK, _CHUNK)],
                    sem,
                )

        def drain_write(g, buf, sem):
            pltpu.make_async_copy(table_hbm.at[pl.ds(0, grows)], buf, sem).wait()
            f = (g * _GROUP) // chunks_per_f
            c = (g * _GROUP) % chunks_per_f
            pltpu.sync_copy(buf, out_hbm.at[f, pl.ds(base + c * _CHUNK, grows)])

        assert chunks_per_f % _GROUP == 0

        fire(0, buf_a, sem_a)

        def body(p, carry):
            g = 2 * p
            fire(g + 1, buf_b, sem_b)
            drain_write(g, buf_a, sem_a)

            @pl.when(p + 1 < steps // (2 * _GROUP))
            def _():
                fire(g + 2, buf_a, sem_a)

            drain_write(g + 1, buf_b, sem_b)
            return carry

        lax.fori_loop(0, steps // (2 * _GROUP), body, 0)

    return gather_kernel


def kernel(x, weight):
    batch, num_fields = x.shape
    num_rows, embed_dim = weight.shape
    tail_rows = num_rows - (num_rows // 128) * 128
    wtail = weight[num_rows - tail_rows:].reshape(tail_rows * embed_dim)
    flat = _build_detile(num_rows, embed_dim)(weight.T, wtail)
    wlin = flat.reshape(num_rows, embed_dim)
    out_t = _build_gather(batch, num_fields, embed_dim)(
        wlin, x.T.astype(jnp.int32)
    )
    return out_t.transpose(1, 0, 2)


# speedup vs baseline: 2.0848x; 1.9653x over previous
import functools

import jax
import jax.numpy as jnp
from jax import lax
from jax.experimental import pallas as pl
from jax.experimental.pallas import tpu as pltpu
from jax.experimental.pallas import tpu_sc as plsc

_NUM_WORKERS = 32
_CHUNK = 128
_GROUP = 4
_LANES = 16


@functools.lru_cache(maxsize=None)
def _build_detile(num_rows: int, embed_dim: int):
    assert embed_dim == 32 and num_rows == 1000000
    jt_n = embed_dim // 8
    full_p = num_rows // 128
    tail = num_rows - full_p * 128
    per_w = full_p // _NUM_WORKERS
    n_left = full_p - per_w * _NUM_WORKERS
    assert per_w % 2 == 0
    mesh = plsc.VectorSubcoreMesh(core_axis_name="c", subcore_axis_name="s")

    @functools.partial(
        pl.kernel,
        out_type=jax.ShapeDtypeStruct((num_rows * embed_dim,), jnp.float32),
        mesh=mesh,
        scratch_types=[
            pltpu.VMEM((jt_n, 8, 128), jnp.float32),
            pltpu.VMEM((jt_n, 8, 128), jnp.float32),
            pltpu.VMEM((128 * embed_dim,), jnp.float32),
            pltpu.VMEM((128 * embed_dim,), jnp.float32),
            pltpu.VMEM((128 * (embed_dim + 1),), jnp.float32),
            pltpu.SemaphoreType.DMA,
            pltpu.SemaphoreType.DMA,
            pltpu.SemaphoreType.DMA,
            pltpu.SemaphoreType.DMA,
        ],
        compiler_params=pltpu.CompilerParams(
            use_tc_tiling_on_sc=True, needs_layout_passes=False,
            disable_bounds_checks=True,
        ),
    )
    def detile_kernel(wt_hbm, wtail_hbm, out_hbm, in_a, in_b, tr_a, tr_b,
                      pad_v, sem_ia, sem_ib, sem_oa, sem_ob):
        num_cores = lax.axis_size("c")
        wid = lax.axis_index("s") * num_cores + lax.axis_index("c")
        lane33 = lax.iota(jnp.int32, _LANES) * (embed_dim + 1)

        def fire_in(p, buf, sem):
            for t in range(jt_n):
                pltpu.async_copy(
                    wt_hbm.at[pl.ds(8 * t, 8), pl.ds(p * 128, 128)],
                    buf.at[t], sem,
                )

        def drain_in(buf, sem):
            for t in range(jt_n):
                pltpu.make_async_copy(
                    wt_hbm.at[pl.ds(0, 8), pl.ds(0, 128)], buf.at[t], sem
                ).wait()

        def transpose(buf, tr, csteps):
            assert csteps == 8

            @plsc.parallel_loop(0, embed_dim * csteps, 1, unroll=8)
            def scat(v):
                m = v >> 3
                c = v & 7
                t = m >> 3
                j = m & 7
                vec = buf[t, j, pl.ds(c * _LANES, _LANES)]
                plsc.store_scatter(
                    pad_v, [lane33 + (c * _LANES * (embed_dim + 1) + m)], vec
                )

            @plsc.parallel_loop(0, embed_dim * csteps, 1, unroll=8)
            def comp(k):
                s = (k << 4) + (k >> 1)
                tr[pl.ds(k * _LANES, _LANES)] = pad_v[pl.ds(s, _LANES)]

        def write_out(p, tr, sem):
            pltpu.async_copy(
                tr, out_hbm.at[pl.ds(p * 128 * embed_dim, 128 * embed_dim)], sem
            )

        def drain_out(tr, sem):
            pltpu.make_async_copy(
                out_hbm.at[pl.ds(0, 128 * embed_dim)], tr, sem
            ).wait()

        fire_in(wid, in_a, sem_ia)

        def body(k, carry):
            p0 = wid + _NUM_WORKERS * 2 * k
            fire_in(p0 + _NUM_WORKERS, in_b, sem_ib)
            drain_in(in_a, sem_ia)

            @pl.when(k > 0)
            def _():
                drain_out(tr_a, sem_oa)

            transpose(in_a, tr_a, 8)
            write_out(p0, tr_a, sem_oa)

            @pl.when(k + 1 < per_w // 2)
            def _():
                fire_in(p0 + 2 * _NUM_WORKERS, in_a, sem_ia)

            drain_in(in_b, sem_ib)

            @pl.when(k > 0)
            def _():
                drain_out(tr_b, sem_ob)

            transpose(in_b, tr_b, 8)
            write_out(p0 + _NUM_WORKERS, tr_b, sem_ob)
            return carry

        lax.fori_loop(0, per_w // 2, body, 0)
        drain_out(tr_a, sem_oa)
        drain_out(tr_b, sem_ob)

        p_extra = wid + _NUM_WORKERS * per_w

        @pl.when(jnp.logical_and(wid < n_left, p_extra < full_p))
        def _():
            fire_in(p_extra, in_a, sem_ia)
            drain_in(in_a, sem_ia)
            transpose(in_a, tr_a, 8)
            write_out(p_extra, tr_a, sem_oa)
            drain_out(tr_a, sem_oa)

        if tail:
            @pl.when(p_extra == full_p)
            def _():
                pltpu.sync_copy(wtail_hbm, tr_a.at[pl.ds(0, tail * embed_dim)])
                pltpu.sync_copy(
                    tr_a.at[pl.ds(0, tail * embed_dim)],
                    out_hbm.at[pl.ds(full_p * 128 * embed_dim, tail * embed_dim)],
                )

    return detile_kernel


@functools.lru_cache(maxsize=None)
def _build_gather(batch: int, num_fields: int, embed_dim: int):
    assert batch % (_NUM_WORKERS * _CHUNK) == 0
    cols_per_w = batch // _NUM_WORKERS
    chunks_per_f = cols_per_w // _CHUNK
    steps = num_fields * chunks_per_f
    assert steps % (2 * _GROUP) == 0
    grows = _GROUP * _CHUNK
    mesh = plsc.VectorSubcoreMesh(core_axis_name="c", subcore_axis_name="s")

    @functools.partial(
        pl.kernel,
        out_type=jax.ShapeDtypeStruct((num_fields, batch, embed_dim), jnp.float32),
        mesh=mesh,
        scratch_types=[
            pltpu.VMEM((num_fields, cols_per_w), jnp.int32),
            pltpu.VMEM((grows, embed_dim), jnp.float32),
            pltpu.VMEM((grows, embed_dim), jnp.float32),
            pltpu.SemaphoreType.DMA,
            pltpu.SemaphoreType.DMA,
        ],
        compiler_params=pltpu.CompilerParams(use_tc_tiling_on_sc=False),
    )
    def gather_kernel(table_hbm, xt_hbm, out_hbm, idx_v, buf_a, buf_b, sem_a, sem_b):
        num_cores = lax.axis_size("c")
        wid = lax.axis_index("s") * num_cores + lax.axis_index("c")
        base = wid * cols_per_w
        pltpu.sync_copy(xt_hbm.at[:, pl.ds(base, cols_per_w)], idx_v)

        def fire(g, buf, sem):
            for k in range(_GROUP):
                s = g * _GROUP + k
                f = s // chunks_per_f
                c = s % chunks_per_f
                pltpu.async_copy(
                    table_hbm.at[idx_v.at[f, pl.ds(c * _CHUNK, _CHUNK)]],
                    buf.at[pl.ds(k * _CHUNK, _CHUNK)],
                    sem,
                )

        def drain_write(g, buf, sem):
            pltpu.make_async_copy(table_hbm.at[pl.ds(0, grows)], buf, sem).wait()
            f = (g * _GROUP) // chunks_per_f
            c = (g * _GROUP) % chunks_per_f
            pltpu.sync_copy(buf, out_hbm.at[f, pl.ds(base + c * _CHUNK, grows)])

        assert chunks_per_f % _GROUP == 0

        fire(0, buf_a, sem_a)

        def body(p, carry):
            g = 2 * p
            fire(g + 1, buf_b, sem_b)
            drain_write(g, buf_a, sem_a)

            @pl.when(p + 1 < steps // (2 * _GROUP))
            def _():
                fire(g + 2, buf_a, sem_a)

            drain_write(g + 1, buf_b, sem_b)
            return carry

        lax.fori_loop(0, steps // (2 * _GROUP), body, 0)

    return gather_kernel


def kernel(x, weight):
    batch, num_fields = x.shape
    num_rows, embed_dim = weight.shape
    tail_rows = num_rows - (num_rows // 128) * 128
    wtail = weight[num_rows - tail_rows:].reshape(tail_rows * embed_dim)
    flat = _build_detile(num_rows, embed_dim)(weight.T, wtail)
    wlin = flat.reshape(num_rows, embed_dim)
    out_t = _build_gather(batch, num_fields, embed_dim)(
        wlin, x.T.astype(jnp.int32)
    )
    return out_t.transpose(1, 0, 2)


# speedup vs baseline: 3.8106x; 1.8278x over previous
import functools

import jax
import jax.numpy as jnp
from jax import lax
from jax.experimental import pallas as pl
from jax.experimental.pallas import tpu as pltpu
from jax.experimental.pallas import tpu_sc as plsc

_NUM_WORKERS = 32
_CHUNK = 128
_GROUP = 4
_LANES = 16


@functools.lru_cache(maxsize=None)
def _build_detile(num_rows: int, embed_dim: int):
    assert embed_dim == 32 and num_rows == 1000000
    jt_n = embed_dim // 8
    full_p = num_rows // 128
    tail = num_rows - full_p * 128
    per_w = full_p // _NUM_WORKERS
    n_left = full_p - per_w * _NUM_WORKERS
    assert per_w % 2 == 0
    mesh = plsc.VectorSubcoreMesh(core_axis_name="c", subcore_axis_name="s")

    @functools.partial(
        pl.kernel,
        out_type=jax.ShapeDtypeStruct((num_rows * embed_dim,), jnp.float32),
        mesh=mesh,
        scratch_types=[
            pltpu.VMEM((jt_n, 8, 128), jnp.float32),
            pltpu.VMEM((jt_n, 8, 128), jnp.float32),
            pltpu.VMEM((128 * embed_dim,), jnp.float32),
            pltpu.VMEM((128 * embed_dim,), jnp.float32),
            pltpu.VMEM((128 * (embed_dim + 1),), jnp.float32),
            pltpu.SemaphoreType.DMA,
            pltpu.SemaphoreType.DMA,
            pltpu.SemaphoreType.DMA,
            pltpu.SemaphoreType.DMA,
        ],
        compiler_params=pltpu.CompilerParams(
            use_tc_tiling_on_sc=True, needs_layout_passes=False,
            disable_bounds_checks=True,
        ),
    )
    def detile_kernel(wt_hbm, wtail_hbm, out_hbm, in_a, in_b, tr_a, tr_b,
                      pad_v, sem_ia, sem_ib, sem_oa, sem_ob):
        num_cores = lax.axis_size("c")
        wid = lax.axis_index("s") * num_cores + lax.axis_index("c")
        lane33 = lax.iota(jnp.int32, _LANES) * (embed_dim + 1)

        def fire_in(p, buf, sem):
            for t in range(jt_n):
                pltpu.async_copy(
                    wt_hbm.at[pl.ds(8 * t, 8), pl.ds(p * 128, 128)],
                    buf.at[t], sem,
                )

        def drain_in(buf, sem):
            for t in range(jt_n):
                pltpu.make_async_copy(
                    wt_hbm.at[pl.ds(0, 8), pl.ds(0, 128)], buf.at[t], sem
                ).wait()

        def transpose(buf, tr, csteps):
            assert csteps == 8

            @plsc.parallel_loop(0, embed_dim * csteps, 1, unroll=8)
            def scat(v):
                m = v >> 3
                c = v & 7
                t = m >> 3
                j = m & 7
                vec = buf[t, j, pl.ds(c * _LANES, _LANES)]
                plsc.store_scatter(
                    pad_v, [lane33 + (c * _LANES * (embed_dim + 1) + m)], vec
                )

            @plsc.parallel_loop(0, embed_dim * csteps, 1, unroll=8)
            def comp(k):
                s = (k << 4) + (k >> 1)
                tr[pl.ds(k * _LANES, _LANES)] = pad_v[pl.ds(s, _LANES)]

        def write_out(p, tr, sem):
            pltpu.async_copy(
                tr, out_hbm.at[pl.ds(p * 128 * embed_dim, 128 * embed_dim)], sem
            )

        def drain_out(tr, sem):
            pltpu.make_async_copy(
                out_hbm.at[pl.ds(0, 128 * embed_dim)], tr, sem
            ).wait()

        fire_in(wid, in_a, sem_ia)

        def body(k, carry):
            p0 = wid + _NUM_WORKERS * 2 * k
            fire_in(p0 + _NUM_WORKERS, in_b, sem_ib)
            drain_in(in_a, sem_ia)

            @pl.when(k > 0)
            def _():
                drain_out(tr_a, sem_oa)

            transpose(in_a, tr_a, 8)
            write_out(p0, tr_a, sem_oa)

            @pl.when(k + 1 < per_w // 2)
            def _():
                fire_in(p0 + 2 * _NUM_WORKERS, in_a, sem_ia)

            drain_in(in_b, sem_ib)

            @pl.when(k > 0)
            def _():
                drain_out(tr_b, sem_ob)

            transpose(in_b, tr_b, 8)
            write_out(p0 + _NUM_WORKERS, tr_b, sem_ob)
            return carry

        lax.fori_loop(0, per_w // 2, body, 0)
        drain_out(tr_a, sem_oa)
        drain_out(tr_b, sem_ob)

        p_extra = wid + _NUM_WORKERS * per_w

        @pl.when(jnp.logical_and(wid < n_left, p_extra < full_p))
        def _():
            fire_in(p_extra, in_a, sem_ia)
            drain_in(in_a, sem_ia)
            transpose(in_a, tr_a, 8)
            write_out(p_extra, tr_a, sem_oa)
            drain_out(tr_a, sem_oa)

        if tail:
            @pl.when(p_extra == full_p)
            def _():
                pltpu.sync_copy(wtail_hbm, tr_a.at[pl.ds(0, tail * embed_dim)])
                pltpu.sync_copy(
                    tr_a.at[pl.ds(0, tail * embed_dim)],
                    out_hbm.at[pl.ds(full_p * 128 * embed_dim, tail * embed_dim)],
                )

    return detile_kernel


@functools.lru_cache(maxsize=None)
def _build_gather(batch: int, num_fields: int, embed_dim: int):
    assert batch % (_NUM_WORKERS * _CHUNK) == 0
    cols_per_w = batch // _NUM_WORKERS
    chunks_per_f = cols_per_w // _CHUNK
    steps = num_fields * chunks_per_f
    assert steps % (2 * _GROUP) == 0
    grows = _GROUP * _CHUNK
    mesh = plsc.VectorSubcoreMesh(core_axis_name="c", subcore_axis_name="s")

    assert chunks_per_f == _GROUP
    jt_n = embed_dim // 8
    pairs = ngroups_pairs = num_fields // 2

    @functools.partial(
        pl.kernel,
        out_type=jax.ShapeDtypeStruct((num_fields * batch * embed_dim,), jnp.float32),
        mesh=mesh,
        scratch_types=[
            pltpu.VMEM((num_fields, cols_per_w), jnp.int32),
            pltpu.VMEM((grows, embed_dim), jnp.float32),
            pltpu.VMEM((grows, embed_dim), jnp.float32),
            pltpu.VMEM((grows * embed_dim,), jnp.float32),
            pltpu.VMEM((grows * embed_dim,), jnp.float32),
            pltpu.VMEM((_CHUNK * (embed_dim + 1),), jnp.float32),
            pltpu.SemaphoreType.DMA,
            pltpu.SemaphoreType.DMA,
            pltpu.SemaphoreType.DMA,
            pltpu.SemaphoreType.DMA,
        ],
        compiler_params=pltpu.CompilerParams(
            use_tc_tiling_on_sc=False, needs_layout_passes=False
        ),
    )
    def gather_kernel(table_hbm, xt_hbm, out_hbm, idx_v, buf_a, buf_b,
                      tr_a, tr_b, pad_v, sem_a, sem_b, sem_wa, sem_wb):
        num_cores = lax.axis_size("c")
        wid = lax.axis_index("s") * num_cores + lax.axis_index("c")
        base = wid * cols_per_w
        lane129 = lax.iota(jnp.int32, _LANES) * (_CHUNK + 1)
        pltpu.sync_copy(xt_hbm.at[:, pl.ds(base, cols_per_w)], idx_v)

        def fire(f, buf, sem):
            for k in range(_GROUP):
                pltpu.async_copy(
                    table_hbm.at[idx_v.at[f, pl.ds(k * _CHUNK, _CHUNK)]],
                    buf.at[pl.ds(k * _CHUNK, _CHUNK)],
                    sem,
                )

        def drain_gather(buf, sem):
            pltpu.make_async_copy(table_hbm.at[pl.ds(0, grows)], buf, sem).wait()

        def transpose_chunk(buf, tr, k):
            @plsc.parallel_loop(0, 2 * _CHUNK, 1, unroll=8)
            def scat(m):
                bi = m >> 1
                h = m & 1
                vec = buf[k * _CHUNK + bi, pl.ds(h * _LANES, _LANES)]
                plsc.store_scatter(
                    pad_v, [lane129 + (h * _LANES * (_CHUNK + 1) + bi)], vec
                )

            @plsc.parallel_loop(0, 2 * _CHUNK, 1, unroll=8)
            def comp(k2):
                s = (k2 << 4) + (k2 >> 3)
                tr[pl.ds(k * _CHUNK * embed_dim + k2 * _LANES, _LANES)] = \
                    pad_v[pl.ds(s, _LANES)]

        def fire_writes(f, tr, sem):
            for k in range(_GROUP):
                bt = wid * _GROUP + k
                for jt in range(jt_n):
                    pltpu.async_copy(
                        tr.at[pl.ds((k * embed_dim + jt * 8) * _CHUNK, 8 * _CHUNK)],
                        out_hbm.at[pl.ds(((f * jt_n + jt) * (batch // _CHUNK) + bt)
                                         * 8 * _CHUNK, 8 * _CHUNK)],
                        sem,
                    )

        def drain_writes(tr, sem):
            pltpu.make_async_copy(out_hbm.at[pl.ds(0, grows * embed_dim)], tr,
                                  sem).wait()

        def process(f, buf, tr, sem, sem_w, first):
            drain_gather(buf, sem)

            @pl.when(jnp.logical_not(first))
            def _():
                drain_writes(tr, sem_w)

            for k in range(_GROUP):
                transpose_chunk(buf, tr, k)
            fire_writes(f, tr, sem_w)

        fire(0, buf_a, sem_a)

        def body(p, carry):
            f = 2 * p
            fire(f + 1, buf_b, sem_b)
            process(f, buf_a, tr_a, sem_a, sem_wa, p == 0)

            @pl.when(p + 1 < pairs)
            def _():
                fire(f + 2, buf_a, sem_a)

            process(f + 1, buf_b, tr_b, sem_b, sem_wb, p == 0)
            return carry

        lax.fori_loop(0, pairs, body, 0)
        drain_writes(tr_a, sem_wa)
        drain_writes(tr_b, sem_wb)

    return gather_kernel


def kernel(x, weight):
    batch, num_fields = x.shape
    num_rows, embed_dim = weight.shape
    tail_rows = num_rows - (num_rows // 128) * 128
    wtail = weight[num_rows - tail_rows:].reshape(tail_rows * embed_dim)
    flat = _build_detile(num_rows, embed_dim)(weight.T, wtail)
    wlin = flat.reshape(num_rows, embed_dim)
    out_t = _build_gather(batch, num_fields, embed_dim)(
        wlin, x.T.astype(jnp.int32)
    )
    return (
        out_t.reshape(num_fields, embed_dim // 8, batch // 128, 8, 128)
        .transpose(2, 4, 0, 1, 3)
        .reshape(batch, num_fields, embed_dim)
    )
